# baseline (speedup 1.0000x reference)
import functools

import jax
import jax.numpy as jnp
from jax import lax
from jax.experimental import pallas as pl
from jax.experimental.pallas import tpu as pltpu
from jax.experimental.pallas import tpu_sc as plsc

NC = 2
NS = 16
NW = NC * NS
NRB = 4
LAG = 2


def _gather_kernel(S, D, src_hbm, table_hbm, out_hbm,
                   idx_v, rowbufs, stagebufs, gsems, ssems):
    wid = lax.axis_index("s") * NC + lax.axis_index("c")
    pltpu.sync_copy(src_hbm.at[:, wid], idx_v)

    iota = lax.iota(jnp.int32, 16)
    cvecs = [jnp.full((16,), c, jnp.int32) for c in range(D)]

    def start_gather(s, b):
        pltpu.async_copy(table_hbm.at[idx_v.at[s // 8, s % 8]],
                         rowbufs[b], gsems[b])

    def wait_gather(b):
        pltpu.make_async_copy(table_hbm.at[idx_v.at[0, 0]],
                              rowbufs[b], gsems[b]).wait()

    def start_store(s, b):
        pltpu.async_copy(stagebufs[b], out_hbm.at[s, :, wid], ssems[b])

    def wait_store(b):
        pltpu.make_async_copy(stagebufs[b], out_hbm.at[0, :, wid],
                              ssems[b]).wait()

    def transpose(b):
        @plsc.parallel_loop(0, 8, unroll=2)
        def _(bb16):
            bbvec = iota + bb16 * 16
            for c in range(D):
                v = plsc.load_gather(rowbufs[b], [bbvec, cvecs[c]])
                stagebufs[b].at[c // 8, c % 8][pl.ds(bb16 * 16, 16)] = v

    @pl.loop(0, S, step=NRB)
    def _(s0):
        for i in range(NRB):
            s = s0 + i
            start_gather(s, i)
            st = s - LAG
            j = (i - LAG) % NRB

            @pl.when(st >= 0)
            def _():
                wait_gather(j)

                @pl.when(st >= NRB)
                def _():
                    wait_store(j)

                transpose(j)
                start_store(st, j)

    for st in range(S - LAG, S):
        j = st % NRB
        wait_gather(j)
        wait_store(j)
        transpose(j)
        start_store(st, j)
    for b in range(NRB):
        wait_store(b)


def kernel(src, table):
    B, S = src.shape
    V, D = table.shape
    BT = B // 128

    src3 = jnp.transpose(src).reshape(S // 8, 8, BT, 128).transpose(0, 2, 1, 3)

    mesh = plsc.VectorSubcoreMesh(core_axis_name="c", subcore_axis_name="s")
    run = functools.partial(
        pl.kernel,
        out_type=jax.ShapeDtypeStruct((S, D // 8, BT, 8, 128), jnp.float32),
        mesh=mesh,
        scratch_types=[
            pltpu.VMEM((S // 8, 8, 128), jnp.int32),
            [pltpu.VMEM((128, D), jnp.float32) for _ in range(NRB)],
            [pltpu.VMEM((D // 8, 8, 128), jnp.float32) for _ in range(NRB)],
            [pltpu.SemaphoreType.DMA for _ in range(NRB)],
            [pltpu.SemaphoreType.DMA for _ in range(NRB)],
        ],
        compiler_params=pltpu.CompilerParams(use_tc_tiling_on_sc=False,
                                             needs_layout_passes=False),
    )(functools.partial(_gather_kernel, S, D))
    y = run(src3, table)

    return jnp.transpose(y, (2, 4, 0, 1, 3)).reshape(B, S, D)
